# Initial kernel scaffold; baseline (speedup 1.0000x reference)
import functools

import jax
import jax.numpy as jnp
from jax import lax
from jax.experimental import pallas as pl
from jax.experimental.pallas import tpu as pltpu
from jax.experimental.pallas import tpu_sc as plsc

F32 = jnp.float32
I32 = jnp.int32
HI = lax.Precision.HIGHEST

B = 2
N = 1024
E = 64
K = 20
BN = B * N
RB = 256
NEDGE = 3 * K * BN

NW = 32
PER_W = NEDGE // NW
CH = 128
NCH = PER_W // CH


def _dot(a, b, dims):
    return lax.dot_general(a, b, (dims, ((), ())), precision=HI,
                           preferred_element_type=F32)


def _prep(read_h, wa_ref, wb_ref, a_ref, bt_ref, idx_ref, d_sc, am_sc):
    wa = wa_ref[...]
    wb = wb_ref[...]
    ones_c = jnp.ones((E, 1), F32)
    ones_r = jnp.ones((1, E), F32)
    hs = [read_h(r) for r in range(3)]
    for r in range(3):
        a_ref[r] = _dot(hs[r], wa, ((1,), (0,)))
        bt_ref[pl.ds(r * BN, BN), :] = _dot(hs[r], wb, ((1,), (0,)))
    pcol = [_dot(hs[r], ones_c, ((1,), (0,))) * (1.0 / E) for r in range(3)]
    prow = [[_dot(ones_r, hs[r][b * N:(b + 1) * N, :], ((1,), (1,)))
             * (1.0 / E) for b in range(B)] for r in range(3)]
    for b in range(B):
        for blk in range(N // RB):
            base = blk * RB
            colio = lax.broadcasted_iota(F32, (RB, N), 1)
            rowio = lax.broadcasted_iota(F32, (RB, N), 0) + float(base)
            lo = b * N + base
            d0 = pcol[0][lo:lo + RB, :] - prow[0][b]
            d1 = pcol[1][lo:lo + RB, :] - prow[1][b]
            d2 = pcol[2][lo:lo + RB, :] - prow[2][b]
            d = (d0 * d0 + d1 * d1) + d2 * d2
            d = d + jnp.where(colio == rowio, F32(1e10), F32(0.0))
            d_sc[...] = d
            tlane = lax.broadcasted_iota(F32, (RB, 128), 1)

            def body(t, carry):
                dv = d_sc[...]
                m = jnp.min(dv, axis=1, keepdims=True)
                w = jnp.where(dv == m, colio, F32(2048.0))
                am = jnp.min(w, axis=1, keepdims=True)
                am_sc[...] = jnp.where(tlane == t.astype(F32), am, am_sc[...])
                d_sc[...] = jnp.where(colio == am, F32(jnp.inf), dv)
                return carry

            lax.fori_loop(0, K, body, 0)
            ios = lax.broadcasted_iota(F32, (RB, RB), 0)
            iol = lax.broadcasted_iota(F32, (RB, RB), 1)
            eye = jnp.where(ios == iol, F32(1.0), F32(0.0))
            amt = _dot(am_sc[...], eye, ((0,), (0,)))
            iv = amt[0:K, :].astype(I32) + (b * N)
            for r in range(3):
                idx_ref[r, :, pl.ds(b * N + base, RB)] = iv + r * BN


def _embed_prep_body(x_ref, w0_ref, wa_ref, wb_ref,
                     hc_ref, a_ref, bt_ref, idx_ref, d_sc, am_sc):
    w0 = w0_ref[...]
    hr = [x_ref[:, r:r + 1] * w0 for r in range(3)]
    nsq = (hr[0] * hr[0] + hr[1] * hr[1]) + hr[2] * hr[2]
    n = jnp.sqrt(nsq)
    s = n / (n + F32(1e-8))
    for r in range(3):
        hc_ref[r] = hr[r] * s
    _prep(lambda r: hc_ref[r], wa_ref, wb_ref, a_ref, bt_ref, idx_ref,
          d_sc, am_sc)


def _prep_body(h_ref, wa_ref, wb_ref, a_ref, bt_ref, idx_ref, d_sc, am_sc):
    _prep(lambda r: h_ref[r], wa_ref, wb_ref, a_ref, bt_ref, idx_ref,
          d_sc, am_sc)


def _maxpool_body(g_ref, a_ref, h_ref, hn_ref):
    h2 = [g_ref[r] + a_ref[r][None, :, :] for r in range(3)]
    nsq = (h2[0] * h2[0] + h2[1] * h2[1]) + h2[2] * h2[2]
    m = jnp.max(nsq, axis=0)
    kio = lax.broadcasted_iota(F32, (K, nsq.shape[1], E), 0)
    sel = jnp.where(nsq == m[None, :, :], kio, F32(1e9))
    kmin = jnp.min(sel, axis=0)
    hot = kio == kmin[None, :, :]
    hs = []
    for r in range(3):
        win = jnp.sum(jnp.where(hot, h2[r], F32(0.0)), axis=0)
        hs.append(h_ref[r] + win)
    nrm = jnp.sqrt((hs[0] * hs[0] + hs[1] * hs[1]) + hs[2] * hs[2])
    m1 = jnp.mean(nrm, axis=1, keepdims=True)
    dev = nrm - m1
    std = jnp.sqrt(jnp.mean(dev * dev, axis=1, keepdims=True)) + F32(1e-8)
    scl = (dev / std) / (nrm + F32(1e-8))
    for r in range(3):
        hn_ref[r] = scl * hs[r]


_sc_mesh = plsc.VectorSubcoreMesh(core_axis_name="c", subcore_axis_name="s")


@functools.partial(
    pl.kernel,
    mesh=_sc_mesh,
    out_type=jax.ShapeDtypeStruct((NEDGE, E), F32),
    scratch_types=[
        pltpu.VMEM((PER_W,), I32),
        pltpu.VMEM((CH, E), F32),
        pltpu.VMEM((CH, E), F32),
        pltpu.SemaphoreType.DMA,
        pltpu.SemaphoreType.DMA,
    ],
)
def _sc_gather(tab_hbm, idx_hbm, out_hbm, idx_v, buf0, buf1, sem0, sem1):
    wid = lax.axis_index("s") * 2 + lax.axis_index("c")
    base_w = wid * PER_W
    pltpu.sync_copy(idx_hbm.at[pl.ds(base_w, PER_W)], idx_v)
    bufs = [buf0, buf1]
    sems = [sem0, sem1]
    cps = [None, None]
    cps[0] = pltpu.async_copy(tab_hbm.at[idx_v.at[pl.ds(0, CH)]],
                              bufs[0], sems[0])
    for t in range(NCH):
        cur = t % 2
        if t + 1 < NCH:
            nxt = (t + 1) % 2
            cps[nxt] = pltpu.async_copy(
                tab_hbm.at[idx_v.at[pl.ds((t + 1) * CH, CH)]],
                bufs[nxt], sems[nxt])
        cps[cur].wait()
        pltpu.sync_copy(bufs[cur], out_hbm.at[pl.ds(base_w + t * CH, CH)])


_embed_prep_call = pl.pallas_call(
    _embed_prep_body,
    out_shape=[
        jax.ShapeDtypeStruct((3, BN, E), F32),
        jax.ShapeDtypeStruct((3, BN, E), F32),
        jax.ShapeDtypeStruct((3 * BN, E), F32),
        jax.ShapeDtypeStruct((3, K, BN), I32),
    ],
    scratch_shapes=[
        pltpu.VMEM((RB, N), F32),
        pltpu.VMEM((RB, 128), F32),
    ],
)

_prep_call = pl.pallas_call(
    _prep_body,
    out_shape=[
        jax.ShapeDtypeStruct((3, BN, E), F32),
        jax.ShapeDtypeStruct((3 * BN, E), F32),
        jax.ShapeDtypeStruct((3, K, BN), I32),
    ],
    scratch_shapes=[
        pltpu.VMEM((RB, N), F32),
        pltpu.VMEM((RB, 128), F32),
    ],
)

_maxpool_call = pl.pallas_call(
    _maxpool_body,
    grid=(BN // RB,),
    in_specs=[
        pl.BlockSpec((3, K, RB, E), lambda p: (0, 0, p, 0)),
        pl.BlockSpec((3, RB, E), lambda p: (0, p, 0)),
        pl.BlockSpec((3, RB, E), lambda p: (0, p, 0)),
    ],
    out_specs=pl.BlockSpec((3, RB, E), lambda p: (0, p, 0)),
    out_shape=jax.ShapeDtypeStruct((3, BN, E), F32),
)


@jax.jit
def kernel(x, W0, W1_0, W2_0, W1_1, W2_1, W1_2, W2_2, W1_3, W2_3):
    W1s = [W1_0, W1_1, W1_2, W1_3]
    W2s = [W2_0, W2_1, W2_2, W2_3]
    was, wbs = [], []
    for W1, W2 in zip(W1s, W2s):
        w1a, w1b = W1[:E], W1[E:]
        was.append(lax.dot_general(w1a - w1b, W2, (((1,), (0,)), ((), ())),
                                   precision=HI))
        wbs.append(lax.dot_general(w1b, W2, (((1,), (0,)), ((), ())),
                                   precision=HI))
    x2 = x.reshape(BN, 3)
    hc, a, bt, idx = _embed_prep_call(x2, W0, was[0], wbs[0])
    for i in range(4):
        g = _sc_gather(bt, idx.reshape(-1))
        hn = _maxpool_call(g.reshape(3, K, BN, E), a, hc)
        if i < 3:
            a, bt, idx = _prep_call(hn, was[i + 1], wbs[i + 1])
        hc = hn
    return hc.reshape(3, B, N, E).transpose(1, 2, 0, 3)


# speedup vs baseline: 1.3396x; 1.3396x over previous
import jax
import jax.numpy as jnp
from jax import lax
from jax.experimental import pallas as pl
from jax.experimental.pallas import tpu as pltpu

F32 = jnp.float32
I32 = jnp.int32
B = 2
N = 1024
E = 64
K = 20
RB = 256


def _fiota(shape, dim):
    return lax.broadcasted_iota(I32, shape, dim).astype(F32)


def _topk_d_body(d_ref, knn_ref, d_sc, am_sc):
    colio = _fiota((RB, N), 1)
    d_sc[...] = d_ref[0]
    tlane = _fiota((RB, 128), 1)

    def body(t, carry):
        dv = d_sc[...]
        m = jnp.min(dv, axis=1, keepdims=True)
        w = jnp.where(dv == m, colio, F32(2048.0))
        am = jnp.min(w, axis=1, keepdims=True)
        am_sc[...] = jnp.where(tlane == t.astype(F32), am, am_sc[...])
        d_sc[...] = jnp.where(colio == am, F32(jnp.inf), dv)
        return carry

    lax.fori_loop(0, K, body, 0)
    knn_ref[0] = am_sc[:, 0:K].astype(I32)


_topk_d = pl.pallas_call(
    _topk_d_body,
    grid=(B, N // RB),
    in_specs=[pl.BlockSpec((1, RB, N), lambda b, p: (b, p, 0))],
    out_specs=pl.BlockSpec((1, RB, K), lambda b, p: (b, p, 0)),
    out_shape=jax.ShapeDtypeStruct((B, N, K), I32),
    scratch_shapes=[pltpu.VMEM((RB, N), F32), pltpu.VMEM((RB, 128), F32)],
)


def _vn_relu(x):
    norm = jnp.linalg.norm(x, axis=2, keepdims=True)
    direction = x / (norm + 1e-08)
    return jax.nn.relu(norm) * direction


def _vn_layernorm(x):
    norm = jnp.linalg.norm(x, axis=2, keepdims=True)
    mean_norm = jnp.mean(norm, axis=-1, keepdims=True)
    std_norm = jnp.std(norm, axis=-1, keepdims=True) + 1e-08
    norm_normalized = (norm - mean_norm) / std_norm
    direction = x / (norm + 1e-08)
    return norm_normalized * direction


def _vn_edge_conv(x, W1, W2, k):
    Bq, Nq, _, D = x.shape
    positions = jnp.mean(x, axis=-1)
    d = jnp.sum((positions[:, :, None, :] - positions[:, None, :, :]) ** 2, axis=-1)
    d = d + jnp.eye(Nq)[None, :, :] * 10000000000.0
    knn_indices = _topk_d(d)
    batch_indices = jnp.arange(Bq)[:, None, None]
    neighbor_features = x[batch_indices, knn_indices]
    x_expanded = x[:, :, None, :, :]
    edge_features = jnp.concatenate(
        [jnp.tile(x_expanded, (1, 1, k, 1, 1)), neighbor_features - x_expanded],
        axis=-1)
    h = edge_features.reshape(Bq * Nq * k, 3, 2 * D) @ W1
    h = _vn_relu(h)
    h = h @ W2
    Dout = W2.shape[1]
    h = h.reshape(Bq, Nq, k, 3, Dout)
    norms = jnp.linalg.norm(h, axis=3)
    max_indices = jnp.argmax(norms, axis=2)
    idx = max_indices[:, :, None, None, :]
    return jnp.take_along_axis(h, idx, axis=2)[:, :, 0]


@jax.jit
def kernel(x, W0, W1_0, W2_0, W1_1, W2_1, W1_2, W2_2, W1_3, W2_3):
    W1s = [W1_0, W1_1, W1_2, W1_3]
    W2s = [W2_0, W2_1, W2_2, W2_3]
    h = x[:, :, :, None] @ W0
    h = _vn_relu(h)
    for i in range(4):
        h_new = _vn_edge_conv(h, W1s[i], W2s[i], K)
        h = h + h_new
        h = _vn_layernorm(h)
    return h
